# baseline (speedup 1.0000x reference)
import functools

import jax
import jax.numpy as jnp
from jax import lax
from jax.experimental import pallas as pl
from jax.experimental.pallas import tpu as pltpu
from jax.experimental.pallas import tpu_sc as plsc

_G = 128
_CH = 256


@functools.lru_cache(maxsize=None)
def _make_gather(V, D, BW, BC):
    info = plsc.get_sparse_core_info()
    NC, NS = info.num_cores, info.num_subcores
    NW = NC * NS

    WPW = BW // NW
    CPW = BC // NW
    w_ch = WPW // _CH
    c_ch = CPW // _CH
    K = 2
    SPC = _CH // _G
    assert WPW * NW == BW and CPW * NW == BC and _CH % _G == 0
    assert w_ch <= K and c_ch % K == 0

    mesh = plsc.VectorSubcoreMesh(core_axis_name="c", subcore_axis_name="s")

    @functools.partial(
        pl.kernel,
        mesh=mesh,
        out_type=[
            jax.ShapeDtypeStruct((BW, D), jnp.float32),
            jax.ShapeDtypeStruct((BC, D), jnp.float32),
        ],
        scratch_types=[
            pltpu.VMEM((WPW,), jnp.int32),
            pltpu.VMEM((CPW,), jnp.int32),
            pltpu.VMEM((_CH, D), jnp.float32),
            pltpu.VMEM((_CH, D), jnp.float32),
            pltpu.SemaphoreType.DMA,
            pltpu.SemaphoreType.DMA,
            pltpu.SemaphoreType.DMA,
            pltpu.SemaphoreType.DMA,
        ],
    )
    def gather_kernel(w_tab, c_tab, widx_hbm, cidx_hbm, w_out, c_out,
                      widx_v, cidx_v, r0, r1, g0, g1, s0, s1):
        wid = lax.axis_index("s") * NC + lax.axis_index("c")
        bufs = (r0, r1)
        gsems = (g0, g1)
        ssems = (s0, s1)

        pltpu.sync_copy(widx_hbm.at[pl.ds(wid * WPW, WPW)], widx_v)
        pltpu.sync_copy(cidx_hbm.at[pl.ds(wid * CPW, CPW)], cidx_v)

        def ring(n_ch, issue_gather, wait_gather, issue_scatter, wait_scatter):
            P = n_ch // K
            for b in range(K - 1):
                issue_gather(b, b)

            def body(p, carry):
                j0 = K * p
                for b in range(K):
                    j = j0 + b
                    wait_gather(b)
                    issue_scatter(j, b)
                    if b == 0:
                        @pl.when(p >= 1)
                        def _():
                            wait_scatter(K - 1)
                        issue_gather(j + K - 1, K - 1)
                    else:
                        @pl.when(p + 1 < P)
                        def _():
                            wait_scatter(b - 1)
                            issue_gather(j + K - 1, b - 1)
                return carry

            lax.fori_loop(0, P, body, 0)
            for b in range(K):
                wait_scatter(b)

        def smallphase(n_ch, issue_gather, wait_gather, issue_scatter, wait_scatter):
            for b in range(n_ch):
                issue_gather(b, b)
            for b in range(n_ch):
                wait_gather(b)
                issue_scatter(b, b)
            for b in range(n_ch):
                wait_scatter(b)

        def phase(tab, idx_v, out, base, n_ch, runner):
            def ig(j, b):
                for s in range(SPC):
                    pltpu.async_copy(
                        tab.at[idx_v.at[pl.ds(j * _CH + s * _G, _G)]],
                        bufs[b].at[pl.ds(s * _G, _G)], gsems[b])

            def wg(b):
                for s in range(SPC):
                    pltpu.make_async_copy(tab.at[idx_v.at[pl.ds(0, _G)]],
                                          bufs[b].at[pl.ds(0, _G)], gsems[b]).wait()

            def isc(j, b):
                pltpu.async_copy(bufs[b], out.at[pl.ds(base + j * _CH, _CH)], ssems[b])

            def wsc(b):
                pltpu.make_async_copy(bufs[b], out.at[pl.ds(base, _CH)], ssems[b]).wait()

            runner(n_ch, ig, wg, isc, wsc)

        phase(w_tab, widx_v, w_out, wid * WPW, w_ch, smallphase)
        phase(c_tab, cidx_v, c_out, wid * CPW, c_ch, ring)

    return gather_kernel


def kernel(words, contexts, w_table, c_table):
    B, = words.shape
    _, L = contexts.shape
    V, D = w_table.shape

    widx = words.astype(jnp.int32)
    cidx = contexts.astype(jnp.int32).T.reshape(B * L)

    w_out, c2d = _make_gather(V, D, B, B * L)(w_table, c_table, widx, cidx)
    return w_out, c2d.reshape(L, B, D).transpose(1, 0, 2)


# speedup vs baseline: 1.0196x; 1.0196x over previous
import functools

import jax
import jax.numpy as jnp
from jax import lax
from jax.experimental import pallas as pl
from jax.experimental.pallas import tpu as pltpu
from jax.experimental.pallas import tpu_sc as plsc

_CH = 128


@functools.lru_cache(maxsize=None)
def _make_gather(V, D, BW, BC):
    info = plsc.get_sparse_core_info()
    NC, NS = info.num_cores, info.num_subcores
    NW = NC * NS

    WPW = BW // NW
    CPW = BC // NW
    w_ch = WPW // _CH
    c_ch = CPW // _CH
    K = 5
    assert WPW * NW == BW and CPW * NW == BC
    assert w_ch == 4 and c_ch % K == 0

    mesh = plsc.VectorSubcoreMesh(core_axis_name="c", subcore_axis_name="s")

    @functools.partial(
        pl.kernel,
        mesh=mesh,
        out_type=[
            jax.ShapeDtypeStruct((BW, D), jnp.float32),
            jax.ShapeDtypeStruct((BC, D), jnp.float32),
        ],
        scratch_types=[
            pltpu.VMEM((WPW,), jnp.int32),
            pltpu.VMEM((CPW,), jnp.int32),
            pltpu.VMEM((_CH, D), jnp.float32),
            pltpu.VMEM((_CH, D), jnp.float32),
            pltpu.VMEM((_CH, D), jnp.float32),
            pltpu.VMEM((_CH, D), jnp.float32),
            pltpu.VMEM((_CH, D), jnp.float32),
            pltpu.VMEM((_CH, D), jnp.float32),
            pltpu.VMEM((_CH, D), jnp.float32),
            pltpu.SemaphoreType.DMA,
            pltpu.SemaphoreType.DMA,
            pltpu.SemaphoreType.DMA,
            pltpu.SemaphoreType.DMA,
            pltpu.SemaphoreType.DMA,
            pltpu.SemaphoreType.DMA,
            pltpu.SemaphoreType.DMA,
            pltpu.SemaphoreType.DMA,
            pltpu.SemaphoreType.DMA,
            pltpu.SemaphoreType.DMA,
            pltpu.SemaphoreType.DMA,
            pltpu.SemaphoreType.DMA,
            pltpu.SemaphoreType.DMA,
            pltpu.SemaphoreType.DMA,
        ],
    )
    def gather_kernel(w_tab, c_tab, widx_hbm, cidx_hbm, w_out, c_out,
                      widx_v, cidx_v, r0, r1, r2, r3, r4, u0, u1,
                      g0, g1, g2, g3, g4, s0, s1, s2, s3, s4,
                      wg0, wg1, ws0, ws1):
        wid = lax.axis_index("s") * NC + lax.axis_index("c")
        bufs = (r0, r1, r2, r3, r4)
        gsems = (g0, g1, g2, g3, g4)
        ssems = (s0, s1, s2, s3, s4)
        wbufs = (u0, u1)
        wgsems = (wg0, wg1)
        wssems = (ws0, ws1)

        pltpu.sync_copy(widx_hbm.at[pl.ds(wid * WPW, WPW)], widx_v)
        pltpu.sync_copy(cidx_hbm.at[pl.ds(wid * CPW, CPW)], cidx_v)

        cbase = wid * CPW
        wbase = wid * WPW

        def c_ig(j, b):
            pltpu.async_copy(c_tab.at[cidx_v.at[pl.ds(j * _CH, _CH)]], bufs[b], gsems[b])

        def c_wg(b):
            pltpu.make_async_copy(c_tab.at[cidx_v.at[pl.ds(0, _CH)]], bufs[b], gsems[b]).wait()

        def c_is(j, b):
            pltpu.async_copy(bufs[b], c_out.at[pl.ds(cbase + j * _CH, _CH)], ssems[b])

        def c_ws(b):
            pltpu.make_async_copy(bufs[b], c_out.at[pl.ds(cbase, _CH)], ssems[b]).wait()

        def w_ig(j, b):
            pltpu.async_copy(w_tab.at[widx_v.at[pl.ds(j * _CH, _CH)]], wbufs[b], wgsems[b])

        def w_wg(b):
            pltpu.make_async_copy(w_tab.at[widx_v.at[pl.ds(0, _CH)]], wbufs[b], wgsems[b]).wait()

        def w_is(j, b):
            pltpu.async_copy(wbufs[b], w_out.at[pl.ds(wbase + j * _CH, _CH)], wssems[b])

        def w_ws(b):
            pltpu.make_async_copy(wbufs[b], w_out.at[pl.ds(wbase, _CH)], wssems[b]).wait()

        for b in range(K - 1):
            c_ig(b, b)

        w_ig(0, 0)
        w_ig(1, 1)
        w_wg(0)
        w_is(0, 0)
        w_wg(1)
        w_is(1, 1)
        w_ws(0)
        w_ig(2, 0)
        w_ws(1)
        w_ig(3, 1)
        w_wg(0)
        w_is(2, 0)
        w_wg(1)
        w_is(3, 1)
        w_ws(0)
        w_ws(1)

        P = c_ch // K

        def body(p, carry):
            j0 = K * p
            for b in range(K):
                j = j0 + b
                c_wg(b)
                c_is(j, b)
                if b == 0:
                    @pl.when(p >= 1)
                    def _():
                        c_ws(K - 1)
                    c_ig(j + K - 1, K - 1)
                else:
                    @pl.when(p + 1 < P)
                    def _():
                        c_ws(b - 1)
                        c_ig(j + K - 1, b - 1)
            return carry

        lax.fori_loop(0, P, body, 0)
        for b in range(K):
            c_ws(b)

    return gather_kernel


def kernel(words, contexts, w_table, c_table):
    B, = words.shape
    _, L = contexts.shape
    V, D = w_table.shape

    widx = words.astype(jnp.int32)
    cidx = contexts.astype(jnp.int32).T.reshape(B * L)

    w_out, c2d = _make_gather(V, D, B, B * L)(w_table, c_table, widx, cidx)
    return w_out, c2d.reshape(L, B, D).transpose(1, 0, 2)


# speedup vs baseline: 1.0216x; 1.0020x over previous
import functools

import jax
import jax.numpy as jnp
from jax import lax
from jax.experimental import pallas as pl
from jax.experimental.pallas import tpu as pltpu
from jax.experimental.pallas import tpu_sc as plsc

_CH = 128


@functools.lru_cache(maxsize=None)
def _make_gather(V, D, BW, BC):
    info = plsc.get_sparse_core_info()
    NC, NS = info.num_cores, info.num_subcores
    NW = NC * NS

    WPW = BW // NW
    CPW = BC // NW
    w_ch = WPW // _CH
    c_ch = CPW // _CH
    K = 5
    assert WPW * NW == BW and CPW * NW == BC
    assert w_ch == 4 and c_ch % K == 0

    mesh = plsc.VectorSubcoreMesh(core_axis_name="c", subcore_axis_name="s")

    @functools.partial(
        pl.kernel,
        mesh=mesh,
        out_type=[
            jax.ShapeDtypeStruct((BW, D), jnp.float32),
            jax.ShapeDtypeStruct((BC, D), jnp.float32),
        ],
        scratch_types=[
            pltpu.VMEM((WPW,), jnp.int32),
            pltpu.VMEM((CPW,), jnp.int32),
            pltpu.VMEM((_CH, D), jnp.float32),
            pltpu.VMEM((_CH, D), jnp.float32),
            pltpu.VMEM((_CH, D), jnp.float32),
            pltpu.VMEM((_CH, D), jnp.float32),
            pltpu.VMEM((_CH, D), jnp.float32),
            pltpu.VMEM((_CH, D), jnp.float32),
            pltpu.VMEM((_CH, D), jnp.float32),
            pltpu.SemaphoreType.DMA,
            pltpu.SemaphoreType.DMA,
            pltpu.SemaphoreType.DMA,
            pltpu.SemaphoreType.DMA,
            pltpu.SemaphoreType.DMA,
            pltpu.SemaphoreType.DMA,
            pltpu.SemaphoreType.DMA,
            pltpu.SemaphoreType.DMA,
            pltpu.SemaphoreType.DMA,
            pltpu.SemaphoreType.DMA,
            pltpu.SemaphoreType.DMA,
            pltpu.SemaphoreType.DMA,
            pltpu.SemaphoreType.DMA,
            pltpu.SemaphoreType.DMA,
        ],
    )
    def gather_kernel(w_tab, c_tab, widx_hbm, cidx_hbm, w_out, c_out,
                      widx_v, cidx_v, r0, r1, r2, r3, r4, u0, u1,
                      g0, g1, g2, g3, g4, s0, s1, s2, s3, s4,
                      wg0, wg1, ws0, ws1):
        wid = lax.axis_index("s") * NC + lax.axis_index("c")
        bufs = (r0, r1, r2, r3, r4)
        gsems = (g0, g1, g2, g3, g4)
        ssems = (s0, s1, s2, s3, s4)
        wbufs = (u0, u1)
        wgsems = (wg0, wg1)
        wssems = (ws0, ws1)

        pltpu.sync_copy(widx_hbm.at[pl.ds(wid * WPW, WPW)], widx_v)
        pltpu.sync_copy(cidx_hbm.at[pl.ds(wid * CPW, CPW)], cidx_v)

        cbase = wid * CPW
        wbase = wid * WPW

        H = _CH // 2

        def c_ig(j, b):
            pltpu.async_copy(c_tab.at[cidx_v.at[pl.ds(j * _CH, H)]],
                             bufs[b].at[pl.ds(0, H)], gsems[b])
            pltpu.async_copy(c_tab.at[cidx_v.at[pl.ds(j * _CH + H, H)]],
                             bufs[b].at[pl.ds(H, H)], gsems[b])

        def c_wg(b):
            pltpu.make_async_copy(c_tab.at[cidx_v.at[pl.ds(0, _CH)]], bufs[b], gsems[b]).wait()

        def c_is(j, b):
            pltpu.async_copy(bufs[b], c_out.at[pl.ds(cbase + j * _CH, _CH)], ssems[b])

        def c_ws(b):
            pltpu.make_async_copy(bufs[b], c_out.at[pl.ds(cbase, _CH)], ssems[b]).wait()

        def w_ig(j, b):
            pltpu.async_copy(w_tab.at[widx_v.at[pl.ds(j * _CH, _CH)]], wbufs[b], wgsems[b])

        def w_wg(b):
            pltpu.make_async_copy(w_tab.at[widx_v.at[pl.ds(0, _CH)]], wbufs[b], wgsems[b]).wait()

        def w_is(j, b):
            pltpu.async_copy(wbufs[b], w_out.at[pl.ds(wbase + j * _CH, _CH)], wssems[b])

        def w_ws(b):
            pltpu.make_async_copy(wbufs[b], w_out.at[pl.ds(wbase, _CH)], wssems[b]).wait()

        for b in range(K - 1):
            c_ig(b, b)

        w_ig(0, 0)
        w_ig(1, 1)
        w_wg(0)
        w_is(0, 0)
        w_wg(1)
        w_is(1, 1)
        w_ws(0)
        w_ig(2, 0)
        w_ws(1)
        w_ig(3, 1)
        w_wg(0)
        w_is(2, 0)
        w_wg(1)
        w_is(3, 1)
        w_ws(0)
        w_ws(1)

        P = c_ch // K

        def body(p, carry):
            j0 = K * p
            for b in range(K):
                j = j0 + b
                c_wg(b)
                c_is(j, b)
                if b == 0:
                    @pl.when(p >= 1)
                    def _():
                        c_ws(K - 1)
                    c_ig(j + K - 1, K - 1)
                else:
                    @pl.when(p + 1 < P)
                    def _():
                        c_ws(b - 1)
                        c_ig(j + K - 1, b - 1)
            return carry

        lax.fori_loop(0, P, body, 0)
        for b in range(K):
            c_ws(b)

    return gather_kernel


def kernel(words, contexts, w_table, c_table):
    B, = words.shape
    _, L = contexts.shape
    V, D = w_table.shape

    widx = words.astype(jnp.int32)
    cidx = contexts.astype(jnp.int32).T.reshape(B * L)

    w_out, c2d = _make_gather(V, D, B, B * L)(w_table, c_table, widx, cidx)
    return w_out, c2d.reshape(L, B, D).transpose(1, 0, 2)


# speedup vs baseline: 1.0230x; 1.0013x over previous
import functools

import jax
import jax.numpy as jnp
from jax import lax
from jax.experimental import pallas as pl
from jax.experimental.pallas import tpu as pltpu
from jax.experimental.pallas import tpu_sc as plsc

_CH = 128


@functools.lru_cache(maxsize=None)
def _make_gather(V, D, BW, BC):
    info = plsc.get_sparse_core_info()
    NC, NS = info.num_cores, info.num_subcores
    NW = NC * NS

    WPW = BW // NW
    CPW = BC // NW
    w_ch = WPW // _CH
    c_ch = CPW // _CH
    K = 5
    assert WPW * NW == BW and CPW * NW == BC
    assert w_ch == 4 and c_ch % K == 0

    mesh = plsc.VectorSubcoreMesh(core_axis_name="c", subcore_axis_name="s")

    @functools.partial(
        pl.kernel,
        mesh=mesh,
        out_type=[
            jax.ShapeDtypeStruct((BW, D), jnp.float32),
            jax.ShapeDtypeStruct((BC, D), jnp.float32),
        ],
        scratch_types=[
            pltpu.VMEM((WPW,), jnp.int32),
            pltpu.VMEM((CPW,), jnp.int32),
            pltpu.VMEM((_CH, D), jnp.float32),
            pltpu.VMEM((_CH, D), jnp.float32),
            pltpu.VMEM((_CH, D), jnp.float32),
            pltpu.VMEM((_CH, D), jnp.float32),
            pltpu.VMEM((_CH, D), jnp.float32),
            pltpu.VMEM((_CH, D), jnp.float32),
            pltpu.VMEM((_CH, D), jnp.float32),
            pltpu.SemaphoreType.DMA,
            pltpu.SemaphoreType.DMA,
            pltpu.SemaphoreType.DMA,
            pltpu.SemaphoreType.DMA,
            pltpu.SemaphoreType.DMA,
            pltpu.SemaphoreType.DMA,
            pltpu.SemaphoreType.DMA,
            pltpu.SemaphoreType.DMA,
            pltpu.SemaphoreType.DMA,
            pltpu.SemaphoreType.DMA,
            pltpu.SemaphoreType.DMA,
            pltpu.SemaphoreType.DMA,
            pltpu.SemaphoreType.DMA,
            pltpu.SemaphoreType.DMA,
        ],
    )
    def gather_kernel(w_tab, c_tab, widx_hbm, cidx_hbm, w_out, c_out,
                      widx_v, cidx_v, r0, r1, r2, r3, r4, u0, u1,
                      g0, g1, g2, g3, g4, s0, s1, s2, s3, s4,
                      wg0, wg1, ws0, ws1):
        wid = lax.axis_index("s") * NC + lax.axis_index("c")
        bufs = (r0, r1, r2, r3, r4)
        gsems = (g0, g1, g2, g3, g4)
        ssems = (s0, s1, s2, s3, s4)
        wbufs = (u0, u1)
        wgsems = (wg0, wg1)
        wssems = (ws0, ws1)

        pltpu.sync_copy(widx_hbm.at[pl.ds(wid * WPW, WPW)], widx_v)
        pltpu.sync_copy(cidx_hbm.at[pl.ds(wid * CPW, CPW)], cidx_v)

        cbase = wid * CPW
        wbase = wid * WPW

        def c_ig(j, b):
            pltpu.async_copy(c_tab.at[cidx_v.at[pl.ds(j * _CH, _CH)]], bufs[b], gsems[b])

        def c_wg(b):
            pltpu.make_async_copy(c_tab.at[cidx_v.at[pl.ds(0, _CH)]], bufs[b], gsems[b]).wait()

        def c_is(j, b):
            pltpu.async_copy(bufs[b], c_out.at[pl.ds(cbase + j * _CH, _CH)], ssems[b])

        def c_ws(b):
            pltpu.make_async_copy(bufs[b], c_out.at[pl.ds(cbase, _CH)], ssems[b]).wait()

        def w_ig(j, b):
            pltpu.async_copy(w_tab.at[widx_v.at[pl.ds(j * _CH, _CH)]], wbufs[b], wgsems[b])

        def w_wg(b):
            pltpu.make_async_copy(w_tab.at[widx_v.at[pl.ds(0, _CH)]], wbufs[b], wgsems[b]).wait()

        def w_is(j, b):
            pltpu.async_copy(wbufs[b], w_out.at[pl.ds(wbase + j * _CH, _CH)], wssems[b])

        def w_ws(b):
            pltpu.make_async_copy(wbufs[b], w_out.at[pl.ds(wbase, _CH)], wssems[b]).wait()

        for b in range(K - 1):
            c_ig(b, b)

        w_ig(0, 0)
        w_ig(1, 1)
        w_wg(0)
        w_is(0, 0)
        w_wg(1)
        w_is(1, 1)
        w_ws(0)
        w_ig(2, 0)
        w_ws(1)
        w_ig(3, 1)
        w_wg(0)
        w_is(2, 0)
        w_wg(1)
        w_is(3, 1)
        w_ws(0)
        w_ws(1)

        P = c_ch // K

        def body(p, carry):
            j0 = K * p
            for b in range(K):
                j = j0 + b
                c_wg(b)
                c_is(j, b)
                if b == 0:
                    @pl.when(p >= 1)
                    def _():
                        c_ws(K - 1)
                    c_ig(j + K - 1, K - 1)
                else:
                    @pl.when(p + 1 < P)
                    def _():
                        c_ws(b - 1)
                        c_ig(j + K - 1, b - 1)
            return carry

        lax.fori_loop(0, P, body, 0)
        for b in range(K):
            c_ws(b)

    return gather_kernel


def kernel(words, contexts, w_table, c_table):
    B, = words.shape
    _, L = contexts.shape
    V, D = w_table.shape

    widx = words.astype(jnp.int32)
    cidx = contexts.astype(jnp.int32).T.reshape(B * L)

    w_out, c2d = _make_gather(V, D, B, B * L)(w_table, c_table, widx, cidx)
    return w_out, c2d.reshape(L, B, D).transpose(1, 0, 2)
